# baseline (speedup 1.0000x reference)
import functools

import jax
import jax.numpy as jnp
from jax import lax
from jax.experimental import pallas as pl
from jax.experimental.pallas import tpu as pltpu
from jax.experimental.pallas import tpu_sc as plsc

NUM_MOVIES = 100000
NUM_USERS = 16384
B = 4096
MF = 64
DEG = 20
DEGP = 24

NC = 2
NS = 16
NW = NC * NS
BW = B // NW
CH = 16
NCH = BW // CH
NBUF = 2


def _sc_gather(user_ids, pos_ids, neg_ids, neighbors_p, movie_emb):
  mesh = plsc.VectorSubcoreMesh(core_axis_name="c", subcore_axis_name="s")

  @functools.partial(
      pl.kernel,
      out_type=(
          jax.ShapeDtypeStruct((B, MF), jnp.float32),
          jax.ShapeDtypeStruct((B, MF), jnp.float32),
          jax.ShapeDtypeStruct((B, MF), jnp.float32),
      ),
      mesh=mesh,
      compiler_params=pltpu.CompilerParams(use_tc_tiling_on_sc=False),
      scratch_types=[
          pltpu.VMEM((BW,), jnp.int32),
          pltpu.VMEM((BW,), jnp.int32),
          pltpu.VMEM((BW,), jnp.int32),
          pltpu.VMEM((BW, DEGP), jnp.int32),
          pltpu.VMEM((BW * DEGP,), jnp.int32),
          pltpu.VMEM((NBUF, CH * DEGP, MF), jnp.float32),
          pltpu.VMEM((BW, MF), jnp.float32),
          pltpu.VMEM((BW, MF), jnp.float32),
          pltpu.VMEM((BW, MF), jnp.float32),
          pltpu.SemaphoreType.DMA,
          pltpu.SemaphoreType.DMA,
          pltpu.SemaphoreType.DMA,
          pltpu.SemaphoreType.DMA,
          pltpu.SemaphoreType.DMA,
      ],
  )
  def k(uid_hbm, pid_hbm, nid_hbm, nbrs_hbm, movies_hbm,
        nsum_hbm, pos_hbm, neg_hbm,
        uid_v, pid_v, nid_v, nbr_v, flat_v, rows_v, acc_v, pos_v, neg_v,
        sem_pos, sem_neg, sem_nbr, sem_r0, sem_r1):
    sems = (sem_r0, sem_r1)
    wid = lax.axis_index("s") * NC + lax.axis_index("c")
    base = wid * BW
    pltpu.sync_copy(uid_hbm.at[pl.ds(base, BW)], uid_v)
    pltpu.sync_copy(pid_hbm.at[pl.ds(base, BW)], pid_v)
    pltpu.sync_copy(nid_hbm.at[pl.ds(base, BW)], nid_v)
    cp_pos = pltpu.async_copy(movies_hbm.at[pid_v], pos_v, sem_pos)
    cp_neg = pltpu.async_copy(movies_hbm.at[nid_v], neg_v, sem_neg)
    pltpu.async_copy(nbrs_hbm.at[uid_v], nbr_v, sem_nbr).wait()

    for u in range(BW):
      flat_v[pl.ds(u * DEGP, 16)] = nbr_v[u, pl.ds(0, 16)]
      flat_v[pl.ds(u * DEGP + 8, 16)] = nbr_v[u, pl.ds(8, 16)]

    def fire(c, b):
      pltpu.async_copy(movies_hbm.at[flat_v.at[pl.ds(c * (CH * DEGP),
                                                     CH * DEGP)]],
                       rows_v.at[b], sems[b])

    def drain_and_reduce(c, b):
      pltpu.make_async_copy(movies_hbm.at[flat_v.at[pl.ds(c * (CH * DEGP),
                                                          CH * DEGP)]],
                            rows_v.at[b], sems[b]).wait()
      for uu in range(CH):
        for g in range(MF // 16):
          s = rows_v[b, uu * DEGP, pl.ds(g * 16, 16)]
          for j in range(1, DEG):
            s = s + rows_v[b, uu * DEGP + j, pl.ds(g * 16, 16)]
          acc_v[c * CH + uu, pl.ds(g * 16, 16)] = s

    for b in range(NBUF):
      fire(b, b)

    def step(i, carry):
      c0 = i * NBUF
      for b in range(NBUF):
        drain_and_reduce(c0 + b, b)

        @pl.when(c0 + b + NBUF < NCH)
        def _():
          fire(c0 + b + NBUF, b)
      return carry

    lax.fori_loop(0, NCH // NBUF, step, 0)

    pltpu.sync_copy(acc_v, nsum_hbm.at[pl.ds(base, BW)])
    cp_pos.wait()
    pltpu.sync_copy(pos_v, pos_hbm.at[pl.ds(base, BW)])
    cp_neg.wait()
    pltpu.sync_copy(neg_v, neg_hbm.at[pl.ds(base, BW)])

  return k(user_ids, pos_ids, neg_ids, neighbors_p, movie_emb)


_TC_BLK = 1024


def _tc_body(users_ref, nsum_ref, pos_ref, neg_ref,
             wu_ref, wm_ref, w0_ref, w1_ref,
             bu_ref, bm_ref, b0_ref, b1_ref,
             out_u_ref, out_p_ref, out_n_ref):
  dn = (((1,), (1,)), ((), ()))
  wm = wm_ref[...]
  bm = bm_ref[...]
  user_e = (lax.dot_general(users_ref[...], wu_ref[...], dn,
                            preferred_element_type=jnp.float32)
            + lax.dot_general(nsum_ref[...] * (1.0 / DEG), wm, dn,
                              preferred_element_type=jnp.float32)
            + bu_ref[...] + bm)
  pos_e = lax.dot_general(pos_ref[...], wm, dn,
                          preferred_element_type=jnp.float32) + bm
  neg_e = lax.dot_general(neg_ref[...], wm, dn,
                          preferred_element_type=jnp.float32) + bm

  w0 = w0_ref[...]
  w1 = w1_ref[...]
  b0 = b0_ref[...]
  b1 = b1_ref[...]

  def trunk(x):
    h = jnp.maximum(lax.dot_general(x, w0, dn,
                                    preferred_element_type=jnp.float32) + b0,
                    0.0)
    return jnp.maximum(lax.dot_general(h, w1, dn,
                                       preferred_element_type=jnp.float32) + b1,
                       0.0)

  out_u_ref[...] = trunk(user_e)
  out_p_ref[...] = trunk(pos_e)
  out_n_ref[...] = trunk(neg_e)


def _tc_dense(users, nsum, pos_raw, neg_raw, W_user, W_movie, W0, W1,
              b_user, b_movie, b0, b1):
  grid = (B // _TC_BLK,)
  row_spec = pl.BlockSpec((_TC_BLK, MF), lambda i: (i, 0))
  w_spec = pl.BlockSpec((64, 64), lambda i: (0, 0))
  b_spec = pl.BlockSpec((1, 64), lambda i: (0, 0))
  return pl.pallas_call(
      _tc_body,
      grid=grid,
      in_specs=[row_spec, row_spec, row_spec, row_spec,
                w_spec, w_spec, w_spec, w_spec,
                b_spec, b_spec, b_spec, b_spec],
      out_specs=[row_spec, row_spec, row_spec],
      out_shape=[jax.ShapeDtypeStruct((B, 64), jnp.float32)] * 3,
  )(users, nsum, pos_raw, neg_raw, W_user, W_movie, W0, W1,
    b_user.reshape(1, 64), b_movie.reshape(1, 64),
    b0.reshape(1, 64), b1.reshape(1, 64))


def kernel(users, pos_movies, neg_movies, user_ids, pos_movie_ids,
           neg_movie_ids, movie_emb, neighbors, W_user, b_user, W_movie,
           b_movie, W0, b0, W1, b1):
  neighbors_p = jnp.pad(neighbors, ((0, 0), (0, DEGP - DEG)))
  nsum, pos_raw, neg_raw = _sc_gather(user_ids, pos_movie_ids, neg_movie_ids,
                                      neighbors_p, movie_emb)
  out_u, out_p, out_n = _tc_dense(users, nsum, pos_raw, neg_raw,
                                  W_user, W_movie, W0, W1,
                                  b_user, b_movie, b0, b1)
  return (out_u, out_p, out_n)


# speedup vs baseline: 3.3909x; 3.3909x over previous
import functools

import jax
import jax.numpy as jnp
from jax import lax
from jax.experimental import pallas as pl
from jax.experimental.pallas import tpu as pltpu
from jax.experimental.pallas import tpu_sc as plsc

NUM_MOVIES = 100000
NUM_USERS = 16384
B = 4096
MF = 64
DEG = 20

NC = 2
NS = 16
NW = NC * NS
BW = B // NW
CH = 16
NCH = BW // CH
NBUF = 4
L = 16
_RECIP20 = 3277


def _sc_gather(user_ids, pos_ids, neg_ids, neighbors_p, movie_emb):
  mesh = plsc.VectorSubcoreMesh(core_axis_name="c", subcore_axis_name="s")

  @functools.partial(
      pl.kernel,
      out_type=(
          jax.ShapeDtypeStruct((B, MF), jnp.float32),
          jax.ShapeDtypeStruct((B, MF), jnp.float32),
          jax.ShapeDtypeStruct((B, MF), jnp.float32),
      ),
      mesh=mesh,
      compiler_params=pltpu.CompilerParams(use_tc_tiling_on_sc=False,
                                           needs_layout_passes=False),
      scratch_types=[
          pltpu.VMEM((BW,), jnp.int32),
          pltpu.VMEM((BW,), jnp.int32),
          pltpu.VMEM((BW,), jnp.int32),
          pltpu.VMEM((BW * DEG,), jnp.int32),
          pltpu.VMEM((BW * DEG,), jnp.int32),
          pltpu.VMEM((NBUF, CH * DEG, MF), jnp.float32),
          pltpu.VMEM((BW, MF), jnp.float32),
          pltpu.VMEM((BW, MF), jnp.float32),
          pltpu.VMEM((BW, MF), jnp.float32),
          pltpu.SemaphoreType.DMA,
          pltpu.SemaphoreType.DMA,
          pltpu.SemaphoreType.DMA,
          pltpu.SemaphoreType.DMA,
          pltpu.SemaphoreType.DMA,
          pltpu.SemaphoreType.DMA,
          pltpu.SemaphoreType.DMA,
      ],
  )
  def k(uid_hbm, pid_hbm, nid_hbm, nbrs_hbm, movies_hbm,
        nsum_hbm, pos_hbm, neg_hbm,
        uid_v, pid_v, nid_v, flat_v, ids_v, rows_v, acc_v, pos_v, neg_v,
        sem_pos, sem_neg, sem_nbr, sem_r0, sem_r1, sem_r2, sem_r3):
    sems = (sem_r0, sem_r1, sem_r2, sem_r3)
    wid = lax.axis_index("s") * NC + lax.axis_index("c")
    base = wid * BW
    pltpu.sync_copy(uid_hbm.at[pl.ds(base, BW)], uid_v)
    pltpu.sync_copy(pid_hbm.at[pl.ds(base, BW)], pid_v)
    pltpu.sync_copy(nid_hbm.at[pl.ds(base, BW)], nid_v)
    cp_pos = pltpu.async_copy(movies_hbm.at[pid_v], pos_v, sem_pos)
    cp_neg = pltpu.async_copy(movies_hbm.at[nid_v], neg_v, sem_neg)
    lane = lax.iota(jnp.int32, L)

    def posgen(i, carry):
      q = lane + i * L
      uq = lax.shift_right_logical(q * _RECIP20, 16)
      jq = q - uq * DEG
      uid = plsc.load_gather(uid_v, [uq])
      flat_v[pl.ds(pl.multiple_of(i * L, L), L)] = uid * DEG + jq
      return carry

    lax.fori_loop(0, BW * DEG // L, posgen, 0)
    pltpu.async_copy(nbrs_hbm.at[flat_v], ids_v, sem_nbr).wait()

    def fire(c, b):
      pltpu.async_copy(movies_hbm.at[ids_v.at[pl.ds(c * (CH * DEG),
                                                    CH * DEG)]],
                       rows_v.at[b], sems[b])

    def drain_and_reduce(c, b):
      pltpu.make_async_copy(movies_hbm.at[ids_v.at[pl.ds(c * (CH * DEG),
                                                         CH * DEG)]],
                            rows_v.at[b], sems[b]).wait()
      def reduce_user(uu, carry):
        r0 = uu * DEG
        for g in range(MF // 16):
          s = rows_v[b, r0, pl.ds(g * 16, 16)]
          for j in range(1, DEG):
            s = s + rows_v[b, r0 + j, pl.ds(g * 16, 16)]
          acc_v[c * CH + uu, pl.ds(g * 16, 16)] = s
        return carry

      lax.fori_loop(0, CH, reduce_user, 0)

    for b in range(NBUF):
      fire(b, b)

    def step(i, carry):
      c0 = i * NBUF
      for b in range(NBUF):
        drain_and_reduce(c0 + b, b)

        @pl.when(c0 + b + NBUF < NCH)
        def _():
          fire(c0 + b + NBUF, b)
      return carry

    lax.fori_loop(0, NCH // NBUF, step, 0)

    pltpu.sync_copy(acc_v, nsum_hbm.at[pl.ds(base, BW)])
    cp_pos.wait()
    pltpu.sync_copy(pos_v, pos_hbm.at[pl.ds(base, BW)])
    cp_neg.wait()
    pltpu.sync_copy(neg_v, neg_hbm.at[pl.ds(base, BW)])

  return k(user_ids, pos_ids, neg_ids, neighbors_p, movie_emb)


_TC_BLK = 1024


def _tc_body(users_ref, nsum_ref, pos_ref, neg_ref,
             wu_ref, wm_ref, w0_ref, w1_ref,
             bu_ref, bm_ref, b0_ref, b1_ref,
             out_u_ref, out_p_ref, out_n_ref):
  dn = (((1,), (1,)), ((), ()))
  wm = wm_ref[...]
  bm = bm_ref[...]
  user_e = (lax.dot_general(users_ref[...], wu_ref[...], dn,
                            preferred_element_type=jnp.float32)
            + lax.dot_general(nsum_ref[...] * (1.0 / DEG), wm, dn,
                              preferred_element_type=jnp.float32)
            + bu_ref[...] + bm)
  pos_e = lax.dot_general(pos_ref[...], wm, dn,
                          preferred_element_type=jnp.float32) + bm
  neg_e = lax.dot_general(neg_ref[...], wm, dn,
                          preferred_element_type=jnp.float32) + bm

  w0 = w0_ref[...]
  w1 = w1_ref[...]
  b0 = b0_ref[...]
  b1 = b1_ref[...]

  def trunk(x):
    h = jnp.maximum(lax.dot_general(x, w0, dn,
                                    preferred_element_type=jnp.float32) + b0,
                    0.0)
    return jnp.maximum(lax.dot_general(h, w1, dn,
                                       preferred_element_type=jnp.float32) + b1,
                       0.0)

  out_u_ref[...] = trunk(user_e)
  out_p_ref[...] = trunk(pos_e)
  out_n_ref[...] = trunk(neg_e)


def _tc_dense(users, nsum, pos_raw, neg_raw, W_user, W_movie, W0, W1,
              b_user, b_movie, b0, b1):
  grid = (B // _TC_BLK,)
  row_spec = pl.BlockSpec((_TC_BLK, MF), lambda i: (i, 0))
  w_spec = pl.BlockSpec((64, 64), lambda i: (0, 0))
  b_spec = pl.BlockSpec((1, 64), lambda i: (0, 0))
  return pl.pallas_call(
      _tc_body,
      grid=grid,
      in_specs=[row_spec, row_spec, row_spec, row_spec,
                w_spec, w_spec, w_spec, w_spec,
                b_spec, b_spec, b_spec, b_spec],
      out_specs=[row_spec, row_spec, row_spec],
      out_shape=[jax.ShapeDtypeStruct((B, 64), jnp.float32)] * 3,
  )(users, nsum, pos_raw, neg_raw, W_user, W_movie, W0, W1,
    b_user.reshape(1, 64), b_movie.reshape(1, 64),
    b0.reshape(1, 64), b1.reshape(1, 64))


def kernel(users, pos_movies, neg_movies, user_ids, pos_movie_ids,
           neg_movie_ids, movie_emb, neighbors, W_user, b_user, W_movie,
           b_movie, W0, b0, W1, b1):
  nsum, pos_raw, neg_raw = _sc_gather(user_ids, pos_movie_ids, neg_movie_ids,
                                      neighbors.reshape(-1), movie_emb)
  out_u, out_p, out_n = _tc_dense(users, nsum, pos_raw, neg_raw,
                                  W_user, W_movie, W0, W1,
                                  b_user, b_movie, b0, b1)
  return (out_u, out_p, out_n)
